# baseline (speedup 1.0000x reference)
import jax
import jax.numpy as jnp
from jax.experimental import pallas as pl
from jax.experimental.pallas import tpu as pltpu

R1 = 41
R2 = 41
N_STREAMS = 5
TILE_H = 160
CROP_H = 96
CROP_W = 256
LOG2E = 1.4426950408889634
LN2 = 0.6931471805599453


def _softplus2_sum(x, acc):
    h = x.shape[0]
    a = 1.0 + jnp.exp2(x * LOG2E)
    n = h // 8
    k = 0
    while k < n:
        if k + 4 <= n:
            p = ((a[8 * k:8 * k + 8] * a[8 * k + 8:8 * k + 16])
                 * (a[8 * k + 16:8 * k + 24] * a[8 * k + 24:8 * k + 32]))
            k += 4
        elif k + 2 <= n:
            p = a[8 * k:8 * k + 8] * a[8 * k + 8:8 * k + 16]
            k += 2
        else:
            p = a[8 * k:8 * k + 8]
            k += 1
        lp = jnp.log2(p)
        acc = lp if acc is None else acc + lp
    return acc


def _lane_fold(acc):
    out = acc[:, 0:128]
    for j in range(1, acc.shape[1] // 128):
        out = out + acc[:, 128 * j:128 * j + 128]
    return out


def _tile_sum(v):
    acc = v[0:8]
    for k in range(1, CROP_H // 8):
        acc = acc + v[8 * k:8 * k + 8]
    return acc[:, 0:128] + acc[:, 128:256]


def kernel(featureMaps, landmarks):
    B, C, H, W = featureMaps.shape
    L = C // 3
    BL = B * L

    Xi = jnp.round(landmarks[:, :, 0] * (H - 1)).astype(jnp.int32).ravel()
    Yi = jnp.round(landmarks[:, :, 1] * (W - 1)).astype(jnp.int32).ravel()
    r0 = jnp.clip(((Xi - R1) // 8) * 8, 0, H - CROP_H)
    q0 = jnp.clip(((Yi - R1) // 128) * 128, 0, W - CROP_W)
    scalars = jnp.concatenate([Xi, Yi, r0, q0])

    mesh = pltpu.create_tensorcore_mesh("core")
    out_init = jnp.zeros((B, L, 4, 8, 128), jnp.float32)

    def state_fn(refs):
        fm_ref, sc_ref, out_ref = refs

        @pl.core_map(mesh)
        def _():
            def scoped(sc_smem, sc_sem, crop_vmem, crop_sems):
                cp = pltpu.make_async_copy(sc_ref, sc_smem, sc_sem)
                cp.start()
                cp.wait()

                def crop_copy(bl, ch, slot):
                    b = bl // L
                    l = bl % L
                    return pltpu.make_async_copy(
                        fm_ref.at[b, ch * L + l,
                                  pl.ds(pl.multiple_of(
                                      sc_smem[2 * BL + bl], 8), CROP_H),
                                  pl.ds(pl.multiple_of(
                                      sc_smem[3 * BL + bl], 128), CROP_W)],
                        crop_vmem.at[slot, ch],
                        crop_sems.at[slot, ch],
                    )

                def crop_stats(bl, slot, acc_ref, j):
                    X = sc_smem[bl].astype(jnp.float32)
                    Y = sc_smem[BL + bl].astype(jnp.float32)
                    rbase = sc_smem[2 * BL + bl].astype(jnp.float32)
                    cbase = sc_smem[3 * BL + bl].astype(jnp.float32)

                    ii = rbase + jax.lax.broadcasted_iota(
                        jnp.int32, (CROP_H, CROP_W), 0).astype(jnp.float32)
                    jj = cbase + jax.lax.broadcasted_iota(
                        jnp.int32, (CROP_H, CROP_W), 1).astype(jnp.float32)
                    dx = X - ii
                    dy = Y - jj
                    inside = (dx * dx + dy * dy) <= float(R1 * R1)

                    inv_r2 = 1.0 / float(R2)
                    l1v = (jnp.abs(crop_vmem[slot, 1] - dx * inv_r2)
                           + jnp.abs(crop_vmem[slot, 2] - dy * inv_r2))
                    acc_ref[j, 0, 1] = _tile_sum(
                        jnp.where(inside, crop_vmem[slot, 0], 0.0))
                    acc_ref[j, 0, 2] = _tile_sum(jnp.where(inside, l1v, 0.0))
                    acc_ref[j, 0, 3] = _tile_sum(jnp.where(inside, 1.0, 0.0))

                def inner(indices, *args):
                    lgs, acc_ref = args[:-1], args[-1]
                    l, = indices
                    slot0 = jax.lax.rem(2 * l, 6)
                    slot1 = slot0 + 1

                    @pl.when(l == 0)
                    def _():
                        for ch in range(3):
                            crop_copy(0, ch, 0).start()
                            crop_copy(L, ch, 1).start()
                            crop_copy(1, ch, 2).start()
                            crop_copy(L + 1, ch, 3).start()

                    @pl.when(l + 2 < L)
                    def _():
                        ns = jax.lax.rem(2 * (l + 2), 6)
                        for ch in range(3):
                            crop_copy(l + 2, ch, ns).start()
                            crop_copy(L + l + 2, ch, ns + 1).start()

                    sp0 = None
                    sp1 = None
                    for lg in lgs:
                        sp0 = _softplus2_sum(lg[0, 0], sp0)
                        sp1 = _softplus2_sum(lg[1, 0], sp1)
                    acc_ref[0, 0, 0] = _lane_fold(sp0)
                    acc_ref[1, 0, 0] = _lane_fold(sp1)

                    for ch in range(3):
                        crop_copy(l, ch, slot0).wait()
                        crop_copy(L + l, ch, slot1).wait()

                    crop_stats(l, slot0, acc_ref, 0)
                    crop_stats(L + l, slot1, acc_ref, 1)

                def stream_spec(k):
                    return pl.BlockSpec(
                        (2, 1, TILE_H, W),
                        lambda l, k=k: (0, l, k, 0),
                    )

                pltpu.emit_pipeline(
                    inner,
                    grid=(L,),
                    in_specs=[stream_spec(k) for k in range(N_STREAMS)],
                    out_specs=[
                        pl.BlockSpec((2, 1, 4, 8, 128),
                                     lambda l: (0, l, 0, 0, 0)),
                    ],
                    core_axis_name="core",
                    dimension_semantics=(
                        pltpu.GridDimensionSemantics.PARALLEL,
                    ),
                    _explicit_indices=True,
                )(*([fm_ref] * N_STREAMS), out_ref)

            pl.run_scoped(
                scoped,
                pltpu.SMEM((4 * BL,), jnp.int32),
                pltpu.SemaphoreType.DMA,
                pltpu.VMEM((6, 3, CROP_H, CROP_W), jnp.float32),
                pltpu.SemaphoreType.DMA((6, 3)),
            )

    _, _, partials = pl.run_state(state_fn)(
        (featureMaps, scalars, out_init))

    partials = partials.reshape(BL, 4, 8, 128)
    sums = jnp.sum(partials, axis=(2, 3))
    sp = sums[:, 0] * LN2
    xm = sums[:, 1]
    l1 = sums[:, 2]
    cnt = sums[:, 3]
    bce = 2.0 * (sp - xm) / float(H * W)
    return jnp.mean(bce + l1 / cnt)


# speedup vs baseline: 1.0090x; 1.0090x over previous
import jax
import jax.numpy as jnp
from jax.experimental import pallas as pl
from jax.experimental.pallas import tpu as pltpu

R1 = 41
R2 = 41
N_STREAMS = 10
TILE_H = 80
CROP_H = 96
CROP_W = 256
LOG2E = 1.4426950408889634
LN2 = 0.6931471805599453


def _softplus2_sum(x, acc):
    h = x.shape[0]
    a = 1.0 + jnp.exp2(x * LOG2E)
    n = h // 8
    k = 0
    while k < n:
        if k + 4 <= n:
            p = ((a[8 * k:8 * k + 8] * a[8 * k + 8:8 * k + 16])
                 * (a[8 * k + 16:8 * k + 24] * a[8 * k + 24:8 * k + 32]))
            k += 4
        elif k + 2 <= n:
            p = a[8 * k:8 * k + 8] * a[8 * k + 8:8 * k + 16]
            k += 2
        else:
            p = a[8 * k:8 * k + 8]
            k += 1
        lp = jnp.log2(p)
        acc = lp if acc is None else acc + lp
    return acc


def _lane_fold(acc):
    out = acc[:, 0:128]
    for j in range(1, acc.shape[1] // 128):
        out = out + acc[:, 128 * j:128 * j + 128]
    return out


def _tile_sum(v):
    acc = v[0:8]
    for k in range(1, CROP_H // 8):
        acc = acc + v[8 * k:8 * k + 8]
    return acc[:, 0:128] + acc[:, 128:256]


def kernel(featureMaps, landmarks):
    B, C, H, W = featureMaps.shape
    L = C // 3
    BL = B * L

    Xi = jnp.round(landmarks[:, :, 0] * (H - 1)).astype(jnp.int32).ravel()
    Yi = jnp.round(landmarks[:, :, 1] * (W - 1)).astype(jnp.int32).ravel()
    r0 = jnp.clip(((Xi - R1) // 8) * 8, 0, H - CROP_H)
    q0 = jnp.clip(((Yi - R1) // 128) * 128, 0, W - CROP_W)
    scalars = jnp.concatenate([Xi, Yi, r0, q0])

    mesh = pltpu.create_tensorcore_mesh("core")
    out_init = jnp.zeros((B, L, 4, 8, 128), jnp.float32)

    def state_fn(refs):
        fm_ref, sc_ref, out_ref = refs

        @pl.core_map(mesh)
        def _():
            def scoped(sc_smem, sc_sem, crop_vmem, crop_sems):
                cp = pltpu.make_async_copy(sc_ref, sc_smem, sc_sem)
                cp.start()
                cp.wait()

                def crop_copy(bl, ch, slot):
                    b = bl // L
                    l = bl % L
                    return pltpu.make_async_copy(
                        fm_ref.at[b, ch * L + l,
                                  pl.ds(pl.multiple_of(
                                      sc_smem[2 * BL + bl], 8), CROP_H),
                                  pl.ds(pl.multiple_of(
                                      sc_smem[3 * BL + bl], 128), CROP_W)],
                        crop_vmem.at[slot, ch],
                        crop_sems.at[slot, ch],
                    )

                def crop_stats(bl, slot, acc_ref, j):
                    X = sc_smem[bl].astype(jnp.float32)
                    Y = sc_smem[BL + bl].astype(jnp.float32)
                    rbase = sc_smem[2 * BL + bl].astype(jnp.float32)
                    cbase = sc_smem[3 * BL + bl].astype(jnp.float32)

                    ii = rbase + jax.lax.broadcasted_iota(
                        jnp.int32, (CROP_H, CROP_W), 0).astype(jnp.float32)
                    jj = cbase + jax.lax.broadcasted_iota(
                        jnp.int32, (CROP_H, CROP_W), 1).astype(jnp.float32)
                    dx = X - ii
                    dy = Y - jj
                    inside = (dx * dx + dy * dy) <= float(R1 * R1)

                    inv_r2 = 1.0 / float(R2)
                    l1v = (jnp.abs(crop_vmem[slot, 1] - dx * inv_r2)
                           + jnp.abs(crop_vmem[slot, 2] - dy * inv_r2))
                    acc_ref[j, 0, 1] = _tile_sum(
                        jnp.where(inside, crop_vmem[slot, 0], 0.0))
                    acc_ref[j, 0, 2] = _tile_sum(jnp.where(inside, l1v, 0.0))
                    acc_ref[j, 0, 3] = _tile_sum(jnp.where(inside, 1.0, 0.0))

                def inner(indices, *args):
                    lgs, acc_ref = args[:-1], args[-1]
                    l, = indices
                    slot0 = jax.lax.rem(2 * l, 6)
                    slot1 = slot0 + 1

                    @pl.when(l == 0)
                    def _():
                        for ch in range(3):
                            crop_copy(0, ch, 0).start()
                            crop_copy(L, ch, 1).start()
                            crop_copy(1, ch, 2).start()
                            crop_copy(L + 1, ch, 3).start()

                    @pl.when(l + 2 < L)
                    def _():
                        ns = jax.lax.rem(2 * (l + 2), 6)
                        for ch in range(3):
                            crop_copy(l + 2, ch, ns).start()
                            crop_copy(L + l + 2, ch, ns + 1).start()

                    sp0 = None
                    sp1 = None
                    for lg in lgs:
                        sp0 = _softplus2_sum(lg[0, 0], sp0)
                        sp1 = _softplus2_sum(lg[1, 0], sp1)
                    acc_ref[0, 0, 0] = _lane_fold(sp0)
                    acc_ref[1, 0, 0] = _lane_fold(sp1)

                    for ch in range(3):
                        crop_copy(l, ch, slot0).wait()
                        crop_copy(L + l, ch, slot1).wait()

                    crop_stats(l, slot0, acc_ref, 0)
                    crop_stats(L + l, slot1, acc_ref, 1)

                def stream_spec(k):
                    return pl.BlockSpec(
                        (2, 1, TILE_H, W),
                        lambda l, k=k: (0, l, k, 0),
                    )

                pltpu.emit_pipeline(
                    inner,
                    grid=(L,),
                    in_specs=[stream_spec(k) for k in range(N_STREAMS)],
                    out_specs=[
                        pl.BlockSpec((2, 1, 4, 8, 128),
                                     lambda l: (0, l, 0, 0, 0)),
                    ],
                    core_axis_name="core",
                    dimension_semantics=(
                        pltpu.GridDimensionSemantics.PARALLEL,
                    ),
                    _explicit_indices=True,
                )(*([fm_ref] * N_STREAMS), out_ref)

            pl.run_scoped(
                scoped,
                pltpu.SMEM((4 * BL,), jnp.int32),
                pltpu.SemaphoreType.DMA,
                pltpu.VMEM((6, 3, CROP_H, CROP_W), jnp.float32),
                pltpu.SemaphoreType.DMA((6, 3)),
            )

    _, _, partials = pl.run_state(state_fn)(
        (featureMaps, scalars, out_init))

    partials = partials.reshape(BL, 4, 8, 128)
    sums = jnp.sum(partials, axis=(2, 3))
    sp = sums[:, 0] * LN2
    xm = sums[:, 1]
    l1 = sums[:, 2]
    cnt = sums[:, 3]
    bce = 2.0 * (sp - xm) / float(H * W)
    return jnp.mean(bce + l1 / cnt)
